# baseline (speedup 1.0000x reference)
import functools

import jax
import jax.numpy as jnp
from jax import lax
from jax.experimental import pallas as pl
from jax.experimental.pallas import tpu as pltpu
from jax.experimental.pallas import tpu_sc as plsc

NUM_NODES = 1000000
EMBED_DIM = 64
BATCH = 16384
HIST = 50

_HG = 56
_BPC = 16


def _make_gather(nw: int):
    b_per_w = BATCH // nw
    n_chunks = b_per_w // _BPC
    n_pairs = n_chunks // 2
    mesh = plsc.VectorSubcoreMesh(core_axis_name="c", subcore_axis_name="s")

    @functools.partial(
        pl.kernel,
        out_type=jax.ShapeDtypeStruct((BATCH, HIST, EMBED_DIM), jnp.float32),
        mesh=mesh,
        scratch_types=[
            pltpu.VMEM((_BPC, _HG), jnp.int32),
            pltpu.VMEM((_BPC, _HG), jnp.int32),
            pltpu.VMEM((_BPC, _HG, EMBED_DIM), jnp.float32),
            pltpu.VMEM((_BPC, _HG, EMBED_DIM), jnp.float32),
            pltpu.SemaphoreType.DMA,
            pltpu.SemaphoreType.DMA,
            pltpu.SemaphoreType.DMA,
            pltpu.SemaphoreType.DMA,
            pltpu.SemaphoreType.DMA,
            pltpu.SemaphoreType.DMA,
        ],
        compiler_params=pltpu.CompilerParams(use_tc_tiling_on_sc=False),
    )
    def gather_kernel(idx_hbm, table_hbm, out_hbm, ibuf_a, ibuf_b,
                      stage_a, stage_b, isem_a, isem_b, gsem_a, gsem_b,
                      osem_a, osem_b):
        nc = lax.axis_size("c")
        wid = lax.axis_index("s") * nc + lax.axis_index("c")
        b_base = wid * b_per_w

        def idx_copy(chunk, ibuf, isem):
            b0 = pl.multiple_of(b_base + chunk * _BPC, _BPC)
            return pltpu.make_async_copy(
                idx_hbm.at[pl.ds(b0, _BPC), :], ibuf, isem)

        def gathers(ibuf, stage, gsem):
            return [
                pltpu.make_async_copy(
                    table_hbm.at[ibuf.at[bb]],
                    stage.at[bb],
                    gsem,
                )
                for bb in range(_BPC)
            ]

        def writeback(chunk, stage, osem):
            b0 = pl.multiple_of(b_base + chunk * _BPC, _BPC)
            return pltpu.make_async_copy(
                stage.at[:, pl.ds(0, HIST)], out_hbm.at[pl.ds(b0, _BPC)], osem)

        idx_copy(0, ibuf_a, isem_a).start()
        idx_copy(1, ibuf_b, isem_b).start()
        idx_copy(0, ibuf_a, isem_a).wait()
        for g in gathers(ibuf_a, stage_a, gsem_a):
            g.start()

        def body(i, carry):
            c0 = 2 * i
            for g in gathers(ibuf_a, stage_a, gsem_a):
                g.wait()

            @pl.when(i + 1 < n_pairs)
            def _():
                idx_copy(c0 + 2, ibuf_a, isem_a).start()

            @pl.when(i >= 1)
            def _():
                writeback(0, stage_b, osem_b).wait()

            idx_copy(0, ibuf_b, isem_b).wait()
            for g in gathers(ibuf_b, stage_b, gsem_b):
                g.start()
            writeback(c0, stage_a, osem_a).start()

            for g in gathers(ibuf_b, stage_b, gsem_b):
                g.wait()

            @pl.when(i + 1 < n_pairs)
            def _():
                idx_copy(c0 + 3, ibuf_b, isem_b).start()

            writeback(0, stage_a, osem_a).wait()

            @pl.when(i + 1 < n_pairs)
            def _():
                idx_copy(0, ibuf_a, isem_a).wait()
                for g in gathers(ibuf_a, stage_a, gsem_a):
                    g.start()

            writeback(c0 + 1, stage_b, osem_b).start()
            return carry

        lax.fori_loop(0, n_pairs, body, 0)
        writeback(0, stage_b, osem_b).wait()

    return gather_kernel


def kernel(nodes, emb_weight):
    info = plsc.get_sparse_core_info()
    nw = info.num_cores * info.num_subcores
    idx2d = jnp.pad(nodes, ((0, 0), (0, _HG - HIST)))
    return _make_gather(nw)(idx2d, emb_weight)


# speedup vs baseline: 2.7108x; 2.7108x over previous
import functools

import jax
import jax.numpy as jnp
from jax import lax
from jax.experimental import pallas as pl
from jax.experimental.pallas import tpu as pltpu
from jax.experimental.pallas import tpu_sc as plsc

NUM_NODES = 1000000
EMBED_DIM = 64
BATCH = 16384
HIST = 50

_B = BATCH * HIST
_LANE = 128
_CH = 512
_PCH = _CH // 2
_GPC = _CH // _LANE
_IPP = 2 * _GPC


def _make_gather(nw: int):
    b_per_w = _B // nw
    p_per_w = b_per_w // 2
    n_chunks = b_per_w // _CH
    n_pairs = n_chunks // 2
    idx_rows_per_w = b_per_w // _LANE
    mesh = plsc.VectorSubcoreMesh(core_axis_name="c", subcore_axis_name="s")

    @functools.partial(
        pl.kernel,
        out_type=jax.ShapeDtypeStruct((_B // 2, _LANE), jnp.float32),
        mesh=mesh,
        scratch_types=[
            pltpu.VMEM((_IPP, _LANE), jnp.int32),
            pltpu.VMEM((_IPP, _LANE), jnp.int32),
            pltpu.VMEM((_CH, EMBED_DIM), jnp.float32),
            pltpu.VMEM((_CH, EMBED_DIM), jnp.float32),
            pltpu.VMEM((_PCH, _LANE), jnp.float32),
            pltpu.SemaphoreType.DMA,
            pltpu.SemaphoreType.DMA,
            pltpu.SemaphoreType.DMA,
            pltpu.SemaphoreType.DMA,
            pltpu.SemaphoreType.DMA,
        ],
        compiler_params=pltpu.CompilerParams(use_tc_tiling_on_sc=False),
    )
    def gather_kernel(idx_hbm, table_hbm, out_hbm, ibuf0, ibuf1, rows0, rows1,
                      packed, isem0, isem1, gsem0, gsem1, osem):
        nc = lax.axis_size("c")
        wid = lax.axis_index("s") * nc + lax.axis_index("c")
        out_base = wid * p_per_w
        idx_base = wid * idx_rows_per_w

        def repack(src):
            def rb(k, carry):
                for m in range(16):
                    packed[2 * k + m // 8, pl.ds((m % 8) * 16, 16)] = (
                        src[4 * k + m // 4, pl.ds((m % 4) * 16, 16)])
                return carry
            lax.fori_loop(0, _PCH // 2, rb, 0)

        def idx_copy(pair, ibuf, isem):
            row0 = pl.multiple_of(idx_base + pair * _IPP, 8)
            return pltpu.make_async_copy(
                idx_hbm.at[pl.ds(row0, _IPP), :], ibuf, isem)

        def gathers(ibuf, half, rows, gsem):
            return [
                pltpu.make_async_copy(
                    table_hbm.at[ibuf.at[half * _GPC + j]],
                    rows.at[pl.ds(j * _LANE, _LANE), :],
                    gsem,
                )
                for j in range(_GPC)
            ]

        def writeback(chunk):
            base = pl.multiple_of(out_base + chunk * _PCH, _PCH)
            return pltpu.make_async_copy(
                packed, out_hbm.at[pl.ds(base, _PCH), :], osem)

        idx_copy(0, ibuf0, isem0).start()
        idx_copy(1, ibuf1, isem1).start()
        idx_copy(0, ibuf0, isem0).wait()
        for c in gathers(ibuf0, 0, rows0, gsem0):
            c.start()

        def body(p, carry):
            pb = p % 2

            def run(ibuf, isem, ibuf_n, isem_n):
                for c in gathers(ibuf, 0, rows0, gsem0):
                    c.wait()

                for c in gathers(ibuf, 1, rows1, gsem1):
                    c.start()

                @pl.when(p >= 1)
                def _():
                    writeback(0).wait()

                repack(rows0)
                writeback(2 * p).start()

                for c in gathers(ibuf, 1, rows1, gsem1):
                    c.wait()

                @pl.when(p + 2 < n_pairs)
                def _():
                    idx_copy(p + 2, ibuf, isem).start()

                @pl.when(p + 1 < n_pairs)
                def _():
                    idx_copy(p + 1, ibuf_n, isem_n).wait()
                    for c in gathers(ibuf_n, 0, rows0, gsem0):
                        c.start()

                writeback(0).wait()
                repack(rows1)
                writeback(2 * p + 1).start()

            @pl.when(pb == 0)
            def _():
                run(ibuf0, isem0, ibuf1, isem1)

            @pl.when(pb == 1)
            def _():
                run(ibuf1, isem1, ibuf0, isem0)

            return carry

        lax.fori_loop(0, n_pairs, body, 0)
        writeback(0).wait()

    return gather_kernel


def kernel(nodes, emb_weight):
    info = plsc.get_sparse_core_info()
    nw = info.num_cores * info.num_subcores
    idx2d = nodes.reshape(_B // _LANE, _LANE)
    out2 = _make_gather(nw)(idx2d, emb_weight)
    return out2.reshape(BATCH, HIST, EMBED_DIM)


# speedup vs baseline: 2.7113x; 1.0002x over previous
import functools

import jax
import jax.numpy as jnp
from jax import lax
from jax.experimental import pallas as pl
from jax.experimental.pallas import tpu as pltpu
from jax.experimental.pallas import tpu_sc as plsc

NUM_NODES = 1000000
EMBED_DIM = 64
BATCH = 16384
HIST = 50

_B = BATCH * HIST
_LANE = 128
_CH = 512
_GPC = _CH // _LANE
_IPP = 2 * _GPC


def _make_gather(nw: int):
    b_per_w = _B // nw
    n_chunks = b_per_w // _CH
    n_pairs = n_chunks // 2
    idx_rows_per_w = b_per_w // _LANE
    mesh = plsc.VectorSubcoreMesh(core_axis_name="c", subcore_axis_name="s")

    @functools.partial(
        pl.kernel,
        out_type=jax.ShapeDtypeStruct((_B, EMBED_DIM), jnp.float32),
        mesh=mesh,
        scratch_types=[
            pltpu.VMEM((_IPP, _LANE), jnp.int32),
            pltpu.VMEM((_IPP, _LANE), jnp.int32),
            pltpu.VMEM((_CH, EMBED_DIM), jnp.float32),
            pltpu.VMEM((_CH, EMBED_DIM), jnp.float32),
            pltpu.SemaphoreType.DMA,
            pltpu.SemaphoreType.DMA,
            pltpu.SemaphoreType.DMA,
            pltpu.SemaphoreType.DMA,
            pltpu.SemaphoreType.DMA,
            pltpu.SemaphoreType.DMA,
        ],
        compiler_params=pltpu.CompilerParams(use_tc_tiling_on_sc=False),
    )
    def gather_kernel(idx_hbm, table_hbm, out_hbm, ibuf0, ibuf1, rows0, rows1,
                      isem0, isem1, gsem0, gsem1, osem0, osem1):
        nc = lax.axis_size("c")
        wid = lax.axis_index("s") * nc + lax.axis_index("c")
        out_base = wid * b_per_w
        idx_base = wid * idx_rows_per_w

        def idx_copy(pair, ibuf, isem):
            row0 = pl.multiple_of(idx_base + pair * _IPP, 8)
            return pltpu.make_async_copy(
                idx_hbm.at[pl.ds(row0, _IPP), :], ibuf, isem)

        def gathers(ibuf, half, rows, gsem):
            return [
                pltpu.make_async_copy(
                    table_hbm.at[ibuf.at[half * _GPC + j]],
                    rows.at[pl.ds(j * _LANE, _LANE), :],
                    gsem,
                )
                for j in range(_GPC)
            ]

        def writeback(chunk, rows, osem):
            base = pl.multiple_of(out_base + chunk * _CH, _CH)
            return pltpu.make_async_copy(
                rows, out_hbm.at[pl.ds(base, _CH), :], osem)

        idx_copy(0, ibuf0, isem0).start()
        idx_copy(1, ibuf1, isem1).start()
        idx_copy(0, ibuf0, isem0).wait()
        for c in gathers(ibuf0, 0, rows0, gsem0):
            c.start()

        def body(p, carry):
            pb = p % 2

            def run(ibuf, isem, ibuf_n, isem_n):
                for c in gathers(ibuf, 0, rows0, gsem0):
                    c.wait()

                @pl.when(p >= 1)
                def _():
                    writeback(0, rows1, osem1).wait()

                for c in gathers(ibuf, 1, rows1, gsem1):
                    c.start()
                writeback(2 * p, rows0, osem0).start()

                for c in gathers(ibuf, 1, rows1, gsem1):
                    c.wait()

                @pl.when(p + 2 < n_pairs)
                def _():
                    idx_copy(p + 2, ibuf, isem).start()

                writeback(0, rows0, osem0).wait()

                @pl.when(p + 1 < n_pairs)
                def _():
                    idx_copy(p + 1, ibuf_n, isem_n).wait()
                    for c in gathers(ibuf_n, 0, rows0, gsem0):
                        c.start()

                writeback(2 * p + 1, rows1, osem1).start()

            @pl.when(pb == 0)
            def _():
                run(ibuf0, isem0, ibuf1, isem1)

            @pl.when(pb == 1)
            def _():
                run(ibuf1, isem1, ibuf0, isem0)

            return carry

        lax.fori_loop(0, n_pairs, body, 0)
        writeback(0, rows1, osem1).wait()

    return gather_kernel


def kernel(nodes, emb_weight):
    info = plsc.get_sparse_core_info()
    nw = info.num_cores * info.num_subcores
    idx2d = nodes.reshape(_B // _LANE, _LANE)
    out = _make_gather(nw)(idx2d, emb_weight)
    return out.reshape(BATCH, HIST, EMBED_DIM)
